# baseline (speedup 1.0000x reference)
import jax
import jax.numpy as jnp
from jax import lax
from jax.experimental import pallas as pl
from jax.experimental.pallas import tpu as pltpu

HIDDEN = 2048
NUM_EXPERTS = 16
TOP_K = 2
BLOCK = 512


def _body(x_ref, w_ref, logits_ref, idx_ref, w_out_ref):
    logits = lax.dot_general(
        w_ref[...], x_ref[...],
        dimension_numbers=(((1,), (1,)), ((), ())),
        preferred_element_type=jnp.float32,
    )
    b = logits.shape[1]
    iota = lax.broadcasted_iota(jnp.int32, (NUM_EXPERTS, b), 0)
    m1 = jnp.max(logits, axis=0, keepdims=True)
    idx1 = jnp.min(jnp.where(logits == m1, iota, NUM_EXPERTS), axis=0, keepdims=True)
    masked = jnp.where(iota == idx1, -jnp.inf, logits)
    m2 = jnp.max(masked, axis=0, keepdims=True)
    idx2 = jnp.min(jnp.where(masked == m2, iota, NUM_EXPERTS), axis=0, keepdims=True)
    e = jnp.exp(m2 - m1)
    w1 = 1.0 / (1.0 + e)
    w2 = 1.0 - w1
    logits_ref[...] = logits
    row = lax.broadcasted_iota(jnp.int32, (TOP_K, b), 0)
    idx_ref[...] = jnp.where(row == 0, idx1, idx2)
    w_out_ref[...] = jnp.where(row == 0, w1, w2)


def kernel(hidden_states, W):
    b, s, h = hidden_states.shape
    x = hidden_states.reshape(-1, h)
    n = x.shape[0]
    grid = (n // BLOCK,)
    logits_t, idx_t, w_t = pl.pallas_call(
        _body,
        grid=grid,
        in_specs=[
            pl.BlockSpec((BLOCK, h), lambda i: (i, 0)),
            pl.BlockSpec((NUM_EXPERTS, h), lambda i: (0, 0)),
        ],
        out_specs=[
            pl.BlockSpec((NUM_EXPERTS, BLOCK), lambda i: (0, i)),
            pl.BlockSpec((TOP_K, BLOCK), lambda i: (0, i)),
            pl.BlockSpec((TOP_K, BLOCK), lambda i: (0, i)),
        ],
        out_shape=[
            jax.ShapeDtypeStruct((NUM_EXPERTS, n), jnp.float32),
            jax.ShapeDtypeStruct((TOP_K, n), jnp.int32),
            jax.ShapeDtypeStruct((TOP_K, n), jnp.float32),
        ],
        compiler_params=pltpu.CompilerParams(
            dimension_semantics=("arbitrary",),
        ),
    )(x, W)
    return logits_t.T, idx_t.T, w_t.T


# speedup vs baseline: 1.1436x; 1.1436x over previous
import jax
import jax.numpy as jnp
from jax import lax
from jax.experimental import pallas as pl
from jax.experimental.pallas import tpu as pltpu

HIDDEN = 2048
NUM_EXPERTS = 16
TOP_K = 2
BLOCK = 1024


def _top2_softmax(logits):
    b = logits.shape[1]
    iota = lax.broadcasted_iota(jnp.int32, (NUM_EXPERTS, b), 0)
    m1 = jnp.max(logits, axis=0, keepdims=True)
    idx1 = jnp.min(jnp.where(logits == m1, iota, NUM_EXPERTS), axis=0, keepdims=True)
    masked = jnp.where(iota == idx1, -jnp.inf, logits)
    m2 = jnp.max(masked, axis=0, keepdims=True)
    idx2 = jnp.min(jnp.where(masked == m2, iota, NUM_EXPERTS), axis=0, keepdims=True)
    e = jnp.exp(m2 - m1)
    w1 = 1.0 / (1.0 + e)
    w2 = 1.0 - w1
    row = lax.broadcasted_iota(jnp.int32, (TOP_K, b), 0)
    idx = jnp.where(row == 0, idx1, idx2)
    w = jnp.where(row == 0, w1, w2)
    return idx, w


def _body(xa_ref, xb_ref, w_ref, logits_ref, idx_ref, w_out_ref):
    wmat = w_ref[...]
    dn = (((1,), (1,)), ((), ()))
    la = lax.dot_general(wmat, xa_ref[...], dimension_numbers=dn,
                         preferred_element_type=jnp.float32)
    lb = lax.dot_general(wmat, xb_ref[...], dimension_numbers=dn,
                         preferred_element_type=jnp.float32)
    ia, wa = _top2_softmax(la)
    ib, wb = _top2_softmax(lb)
    logits_ref[:, :BLOCK] = la
    logits_ref[:, BLOCK:] = lb
    idx_ref[:, :BLOCK] = ia
    idx_ref[:, BLOCK:] = ib
    w_out_ref[:, :BLOCK] = wa
    w_out_ref[:, BLOCK:] = wb


def kernel(hidden_states, W):
    b, s, h = hidden_states.shape
    x = hidden_states.reshape(-1, h)
    n = x.shape[0]
    grid = (n // (2 * BLOCK),)
    logits_t, idx_t, w_t = pl.pallas_call(
        _body,
        grid=grid,
        in_specs=[
            pl.BlockSpec((BLOCK, h), lambda i: (2 * i, 0)),
            pl.BlockSpec((BLOCK, h), lambda i: (2 * i + 1, 0)),
            pl.BlockSpec((NUM_EXPERTS, h), lambda i: (0, 0)),
        ],
        out_specs=[
            pl.BlockSpec((NUM_EXPERTS, 2 * BLOCK), lambda i: (0, i)),
            pl.BlockSpec((TOP_K, 2 * BLOCK), lambda i: (0, i)),
            pl.BlockSpec((TOP_K, 2 * BLOCK), lambda i: (0, i)),
        ],
        out_shape=[
            jax.ShapeDtypeStruct((NUM_EXPERTS, n), jnp.float32),
            jax.ShapeDtypeStruct((TOP_K, n), jnp.int32),
            jax.ShapeDtypeStruct((TOP_K, n), jnp.float32),
        ],
        compiler_params=pltpu.CompilerParams(
            dimension_semantics=("arbitrary",),
        ),
    )(x, x, W)
    return logits_t.T, idx_t.T, w_t.T
